# baseline (speedup 1.0000x reference)
import functools

import jax
import jax.numpy as jnp
from jax import lax
from jax.experimental import pallas as pl
from jax.experimental.pallas import tpu as pltpu
from jax.experimental.pallas import tpu_sc as plsc

N = 10000
E = 160000
F_IN = 400
F_HID = 128
NC = 2
NS = 16
DH = F_HID // NC

KD = 40
CHD = E // (NC * NS * KD)
KR = 80
CHR = E // (NS * KR)
GRP = 5

RPT = N // NS
ZR = 125

_mesh = plsc.VectorSubcoreMesh(core_axis_name="c", subcore_axis_name="s")
_sc_params = pltpu.CompilerParams(use_tc_tiling_on_sc=False)


def _zero_fill(zbuf, nrows, width):
    zv = jnp.zeros((16,), jnp.float32)

    def zrow(r, carry):
        for c in range(width // 16):
            zbuf[r, pl.ds(c * 16, 16)] = zv
        return carry

    lax.fori_loop(0, nrows, zrow, 0)


SLAB = 624


@functools.partial(
    pl.kernel,
    out_type=jax.ShapeDtypeStruct((NC, N), jnp.float32),
    mesh=_mesh,
    scratch_types=[
        pltpu.VMEM((CHD, KD), jnp.int32),
        pltpu.VMEM((KD,), jnp.float32),
        pltpu.VMEM((SLAB,), jnp.float32),
        pltpu.VMEM_SHARED((N,), jnp.float32),
    ],
    compiler_params=_sc_params,
)
def _deg_call(dst_hbm, out_hbm, idx_v, ones_v, zbuf, acc_sh):
    cid = lax.axis_index("c")
    sid = lax.axis_index("s")
    zv = jnp.zeros((16,), jnp.float32)
    for o in range(0, SLAB, 16):
        zbuf[pl.ds(o, 16)] = zv
    pltpu.sync_copy(zbuf, acc_sh.at[pl.ds(sid * SLAB, SLAB)])

    @pl.when(sid == 0)
    def _():
        pltpu.sync_copy(zbuf.at[pl.ds(0, 16)],
                        acc_sh.at[pl.ds(NS * SLAB, N - NS * SLAB)])

    ov = jnp.full((16,), 1.0, jnp.float32)
    for o in range(0, KD - 15, 16):
        ones_v[pl.ds(o, 16)] = ov
    ones_v[pl.ds(KD - 16, 16)] = ov
    pltpu.sync_copy(dst_hbm.at[cid, sid], idx_v)
    plsc.subcore_barrier()

    def body(j, carry):
        pltpu.sync_copy(ones_v, acc_sh.at[idx_v.at[j]], add=True)
        return carry

    lax.fori_loop(0, CHD, body, 0)
    plsc.subcore_barrier()
    pltpu.sync_copy(acc_sh.at[pl.ds(sid * SLAB, SLAB)],
                    out_hbm.at[cid, pl.ds(sid * SLAB, SLAB)])

    @pl.when(sid == 0)
    def _():
        pltpu.sync_copy(acc_sh.at[pl.ds(NS * SLAB, N - NS * SLAB)],
                        out_hbm.at[cid, pl.ds(NS * SLAB, N - NS * SLAB)])


@functools.partial(
    pl.kernel,
    out_type=jax.ShapeDtypeStruct((N, NC, DH), jnp.float32),
    mesh=_mesh,
    scratch_types=[
        pltpu.VMEM((CHR, KR), jnp.int32),
        pltpu.VMEM((CHR, KR), jnp.int32),
        pltpu.VMEM((GRP, KR, DH), jnp.float32),
        pltpu.VMEM((ZR, DH), jnp.float32),
        pltpu.VMEM_SHARED((N, DH), jnp.float32),
    ] + [pltpu.SemaphoreType.DMA] * (2 * GRP),
    compiler_params=_sc_params,
)
def _scatter_call(t2_hbm, idx2_hbm, dst_hbm, out_hbm,
                  src_v, dst_v, rows_v, zbuf, acc_sh, *sems):
    cid = lax.axis_index("c")
    sid = lax.axis_index("s")
    _zero_fill(zbuf, ZR, DH)
    for q in range(RPT // ZR):
        pltpu.sync_copy(zbuf, acc_sh.at[pl.ds(sid * RPT + q * ZR, ZR)])
    pltpu.sync_copy(idx2_hbm.at[cid, sid], src_v)
    pltpu.sync_copy(dst_hbm.at[sid], dst_v)
    plsc.subcore_barrier()

    def body(p, carry):
        j = GRP * p
        gathers = [
            pltpu.async_copy(t2_hbm.at[src_v.at[j + i]], rows_v.at[i], sems[i])
            for i in range(GRP)
        ]
        scatters = []
        for i in range(GRP):
            gathers[i].wait()
            scatters.append(
                pltpu.async_copy(rows_v.at[i], acc_sh.at[dst_v.at[j + i]],
                                 sems[GRP + i], add=True))
        for i in range(GRP):
            scatters[i].wait()
        return carry

    lax.fori_loop(0, CHR // GRP, body, 0)
    plsc.subcore_barrier()
    pltpu.sync_copy(acc_sh.at[pl.ds(sid * RPT, RPT)],
                    out_hbm.at[pl.ds(sid * RPT, RPT), cid])


BN = 1000


def _dinv_of(deg0_ref, deg1_ref):
    return lax.rsqrt(deg0_ref[:] + deg1_ref[:] + 1.0)


def _mm1_body(x_ref, w_ref, deg0_ref, deg1_ref, o_ref):
    dinv = _dinv_of(deg0_ref, deg1_ref)
    o_ref[:] = jnp.dot(x_ref[:], w_ref[:],
                       preferred_element_type=jnp.float32) * dinv


def _mid_body(acc_ref, t_ref, deg0_ref, deg1_ref, b_ref, w_ref, o_ref):
    dinv = _dinv_of(deg0_ref, deg1_ref)
    h = jnp.maximum(dinv * (acc_ref[:] + t_ref[:]) + b_ref[:], 0.0)
    o_ref[:] = jnp.dot(h, w_ref[:], preferred_element_type=jnp.float32) * dinv


def _final_body(acc_ref, t_ref, deg0_ref, deg1_ref, b_ref, o_ref):
    dinv = _dinv_of(deg0_ref, deg1_ref)
    o_ref[:] = jnp.maximum(dinv * (acc_ref[:] + t_ref[:]) + b_ref[:], 0.0)


_t_spec = pl.BlockSpec((BN, F_HID), lambda i: (i, 0))
_deg_spec = pl.BlockSpec((BN, 1), lambda i: (i, 0))
_bias_spec = pl.BlockSpec((1, F_HID), lambda i: (0, 0))

_mm1 = pl.pallas_call(
    _mm1_body,
    grid=(N // BN,),
    in_specs=[
        pl.BlockSpec((BN, F_IN), lambda i: (i, 0)),
        pl.BlockSpec((F_IN, F_HID), lambda i: (0, 0)),
        _deg_spec,
        _deg_spec,
    ],
    out_specs=_t_spec,
    out_shape=jax.ShapeDtypeStruct((N, F_HID), jnp.float32),
)

_mid = pl.pallas_call(
    _mid_body,
    grid=(N // BN,),
    in_specs=[
        _t_spec,
        _t_spec,
        _deg_spec,
        _deg_spec,
        _bias_spec,
        pl.BlockSpec((F_HID, F_HID), lambda i: (0, 0)),
    ],
    out_specs=_t_spec,
    out_shape=jax.ShapeDtypeStruct((N, F_HID), jnp.float32),
)

_final = pl.pallas_call(
    _final_body,
    grid=(N // BN,),
    in_specs=[_t_spec, _t_spec, _deg_spec, _deg_spec, _bias_spec],
    out_specs=_t_spec,
    out_shape=jax.ShapeDtypeStruct((N, F_HID), jnp.float32),
)


def kernel(x, edge_index, batch, W1, b1, W2, b2):
    src = edge_index[0]
    dst = edge_index[1]
    dst_deg = dst.reshape(NC, NS, CHD, KD)
    src2 = 2 * src.reshape(NS, CHR, KR)
    idx2 = jnp.stack([src2, src2 + 1])
    dst_r = dst.reshape(NS, CHR, KR)
    b1r = b1.reshape(1, F_HID)
    b2r = b2.reshape(1, F_HID)

    degp = _deg_call(dst_deg)
    deg0 = degp[0].reshape(N, 1)
    deg1 = degp[1].reshape(N, 1)
    t1 = _mm1(x, W1, deg0, deg1)
    acc1 = _scatter_call(t1.reshape(2 * N, DH), idx2, dst_r).reshape(N, F_HID)
    t2 = _mid(acc1, t1, deg0, deg1, b1r, W2)
    acc2 = _scatter_call(t2.reshape(2 * N, DH), idx2, dst_r).reshape(N, F_HID)
    return _final(acc2, t2, deg0, deg1, b2r)


# speedup vs baseline: 1.3202x; 1.3202x over previous
import functools

import jax
import jax.numpy as jnp
from jax import lax
from jax.experimental import pallas as pl
from jax.experimental.pallas import tpu as pltpu
from jax.experimental.pallas import tpu_sc as plsc

N = 10000
E = 160000
F_IN = 400
F_HID = 128
NC = 2
NS = 16
DH = F_HID // NC

KD = 40
CHD = E // (NC * NS * KD)
KR = 80
CHR = E // (NS * KR)
GRP = 5

RPT = N // NS
ZR = 125

_mesh = plsc.VectorSubcoreMesh(core_axis_name="c", subcore_axis_name="s")
_sc_params = pltpu.CompilerParams(use_tc_tiling_on_sc=False)


def _zero_fill(zbuf, nrows, width):
    zv = jnp.zeros((16,), jnp.float32)

    def zrow(r, carry):
        for c in range(width // 16):
            zbuf[r, pl.ds(c * 16, 16)] = zv
        return carry

    lax.fori_loop(0, nrows, zrow, 0)


SLAB = 624


@functools.partial(
    pl.kernel,
    out_type=jax.ShapeDtypeStruct((NC, N), jnp.float32),
    mesh=_mesh,
    scratch_types=[
        pltpu.VMEM((CHD, KD), jnp.int32),
        pltpu.VMEM((KD,), jnp.float32),
        pltpu.VMEM((SLAB,), jnp.float32),
        pltpu.VMEM_SHARED((N,), jnp.float32),
    ],
    compiler_params=_sc_params,
)
def _deg_call(dst_hbm, out_hbm, idx_v, ones_v, zbuf, acc_sh):
    cid = lax.axis_index("c")
    sid = lax.axis_index("s")
    zv = jnp.zeros((16,), jnp.float32)
    for o in range(0, SLAB, 16):
        zbuf[pl.ds(o, 16)] = zv
    pltpu.sync_copy(zbuf, acc_sh.at[pl.ds(sid * SLAB, SLAB)])

    @pl.when(sid == 0)
    def _():
        pltpu.sync_copy(zbuf.at[pl.ds(0, 16)],
                        acc_sh.at[pl.ds(NS * SLAB, N - NS * SLAB)])

    ov = jnp.full((16,), 1.0, jnp.float32)
    for o in range(0, KD - 15, 16):
        ones_v[pl.ds(o, 16)] = ov
    ones_v[pl.ds(KD - 16, 16)] = ov
    pltpu.sync_copy(dst_hbm.at[cid, sid], idx_v)
    plsc.subcore_barrier()

    def body(j, carry):
        pltpu.sync_copy(ones_v, acc_sh.at[idx_v.at[j]], add=True)
        return carry

    lax.fori_loop(0, CHD, body, 0)
    plsc.subcore_barrier()
    pltpu.sync_copy(acc_sh.at[pl.ds(sid * SLAB, SLAB)],
                    out_hbm.at[cid, pl.ds(sid * SLAB, SLAB)])

    @pl.when(sid == 0)
    def _():
        pltpu.sync_copy(acc_sh.at[pl.ds(NS * SLAB, N - NS * SLAB)],
                        out_hbm.at[cid, pl.ds(NS * SLAB, N - NS * SLAB)])


@functools.partial(
    pl.kernel,
    out_type=jax.ShapeDtypeStruct((N, F_HID), jnp.float32),
    mesh=_mesh,
    scratch_types=[
        pltpu.VMEM((CHR, KR), jnp.int32),
        pltpu.VMEM((CHR, KR), jnp.int32),
        pltpu.VMEM((GRP, KR, DH), jnp.float32),
        pltpu.VMEM((ZR, DH), jnp.float32),
        pltpu.VMEM_SHARED((N, DH), jnp.float32),
    ] + [pltpu.SemaphoreType.DMA] * (2 * GRP),
    compiler_params=_sc_params,
)
def _scatter_call(t2_hbm, idx2_hbm, dst_hbm, out_hbm,
                  src_v, dst_v, rows_v, zbuf, acc_sh, *sems):
    cid = lax.axis_index("c")
    sid = lax.axis_index("s")
    _zero_fill(zbuf, ZR, DH)
    for q in range(RPT // ZR):
        pltpu.sync_copy(zbuf, acc_sh.at[pl.ds(sid * RPT + q * ZR, ZR)])
    pltpu.sync_copy(idx2_hbm.at[cid, sid], src_v)
    pltpu.sync_copy(dst_hbm.at[sid], dst_v)
    plsc.subcore_barrier()

    def body(p, carry):
        j = GRP * p
        gathers = [
            pltpu.async_copy(t2_hbm.at[src_v.at[j + i]], rows_v.at[i], sems[i])
            for i in range(GRP)
        ]
        scatters = []
        for i in range(GRP):
            gathers[i].wait()
            scatters.append(
                pltpu.async_copy(rows_v.at[i], acc_sh.at[dst_v.at[j + i]],
                                 sems[GRP + i], add=True))
        for i in range(GRP):
            scatters[i].wait()
        return carry

    lax.fori_loop(0, CHR // GRP, body, 0)
    plsc.subcore_barrier()
    pltpu.sync_copy(acc_sh.at[pl.ds(sid * RPT, RPT)],
                    out_hbm.at[pl.ds(sid * RPT, RPT), pl.ds(cid * DH, DH)])


BN = 1000


def _dinv_of(deg0_ref, deg1_ref):
    return lax.rsqrt(deg0_ref[:] + deg1_ref[:] + 1.0)


def _mm1_body(x_ref, w_ref, deg0_ref, deg1_ref, o_ref):
    dinv = _dinv_of(deg0_ref, deg1_ref)
    o_ref[:] = jnp.dot(x_ref[:], w_ref[:],
                       preferred_element_type=jnp.float32) * dinv


def _mid_body(acc_ref, t_ref, deg0_ref, deg1_ref, b_ref, w_ref, o_ref):
    dinv = _dinv_of(deg0_ref, deg1_ref)
    h = jnp.maximum(dinv * (acc_ref[:] + t_ref[:]) + b_ref[:], 0.0)
    o_ref[:] = jnp.dot(h, w_ref[:], preferred_element_type=jnp.float32) * dinv


def _final_body(acc_ref, t_ref, deg0_ref, deg1_ref, b_ref, o_ref):
    dinv = _dinv_of(deg0_ref, deg1_ref)
    o_ref[:] = jnp.maximum(dinv * (acc_ref[:] + t_ref[:]) + b_ref[:], 0.0)


_t_spec = pl.BlockSpec((BN, F_HID), lambda i: (i, 0))
_deg_spec = pl.BlockSpec((BN, 1), lambda i: (i, 0))
_bias_spec = pl.BlockSpec((1, F_HID), lambda i: (0, 0))

_mm1 = pl.pallas_call(
    _mm1_body,
    grid=(N // BN,),
    in_specs=[
        pl.BlockSpec((BN, F_IN), lambda i: (i, 0)),
        pl.BlockSpec((F_IN, F_HID), lambda i: (0, 0)),
        _deg_spec,
        _deg_spec,
    ],
    out_specs=_t_spec,
    out_shape=jax.ShapeDtypeStruct((N, F_HID), jnp.float32),
)

_mid = pl.pallas_call(
    _mid_body,
    grid=(N // BN,),
    in_specs=[
        _t_spec,
        _t_spec,
        _deg_spec,
        _deg_spec,
        _bias_spec,
        pl.BlockSpec((F_HID, F_HID), lambda i: (0, 0)),
    ],
    out_specs=_t_spec,
    out_shape=jax.ShapeDtypeStruct((N, F_HID), jnp.float32),
)

_final = pl.pallas_call(
    _final_body,
    grid=(N // BN,),
    in_specs=[_t_spec, _t_spec, _deg_spec, _deg_spec, _bias_spec],
    out_specs=_t_spec,
    out_shape=jax.ShapeDtypeStruct((N, F_HID), jnp.float32),
)


def kernel(x, edge_index, batch, W1, b1, W2, b2):
    src = edge_index[0]
    dst = edge_index[1]
    dst_deg = dst.reshape(NC, NS, CHD, KD)
    src2 = 2 * src.reshape(NS, CHR, KR)
    idx2 = jnp.stack([src2, src2 + 1])
    dst_r = dst.reshape(NS, CHR, KR)
    b1r = b1.reshape(1, F_HID)
    b2r = b2.reshape(1, F_HID)

    degp = _deg_call(dst_deg)
    deg0 = degp[0].reshape(N, 1)
    deg1 = degp[1].reshape(N, 1)
    t1 = _mm1(x, W1, deg0, deg1)
    acc1 = _scatter_call(t1.reshape(2 * N, DH), idx2, dst_r)
    t2 = _mid(acc1, t1, deg0, deg1, b1r, W2)
    acc2 = _scatter_call(t2.reshape(2 * N, DH), idx2, dst_r)
    return _final(acc2, t2, deg0, deg1, b2r)


# speedup vs baseline: 1.4375x; 1.0888x over previous
import functools

import jax
import jax.numpy as jnp
from jax import lax
from jax.experimental import pallas as pl
from jax.experimental.pallas import tpu as pltpu
from jax.experimental.pallas import tpu_sc as plsc

N = 10000
E = 160000
F_IN = 400
F_HID = 128
NC = 2
NS = 16
DH = F_HID // NC

KD = 40
CHD = E // (NC * NS * KD)
KR = 80
CHR = E // (NS * KR)
GRP = 10
GTAIL = CHR % GRP

RPT = N // NS
ZR = 125

_mesh = plsc.VectorSubcoreMesh(core_axis_name="c", subcore_axis_name="s")
_sc_params = pltpu.CompilerParams(use_tc_tiling_on_sc=False)


def _zero_fill(zbuf, nrows, width):
    zv = jnp.zeros((16,), jnp.float32)

    def zrow(r, carry):
        for c in range(width // 16):
            zbuf[r, pl.ds(c * 16, 16)] = zv
        return carry

    lax.fori_loop(0, nrows, zrow, 0)


SLAB = 624


@functools.partial(
    pl.kernel,
    out_type=jax.ShapeDtypeStruct((NC, N), jnp.float32),
    mesh=_mesh,
    scratch_types=[
        pltpu.VMEM((CHD, KD), jnp.int32),
        pltpu.VMEM((KD,), jnp.float32),
        pltpu.VMEM((SLAB,), jnp.float32),
        pltpu.VMEM_SHARED((N,), jnp.float32),
    ],
    compiler_params=_sc_params,
)
def _deg_call(dst_hbm, out_hbm, idx_v, ones_v, zbuf, acc_sh):
    cid = lax.axis_index("c")
    sid = lax.axis_index("s")
    zv = jnp.zeros((16,), jnp.float32)
    for o in range(0, SLAB, 16):
        zbuf[pl.ds(o, 16)] = zv
    pltpu.sync_copy(zbuf, acc_sh.at[pl.ds(sid * SLAB, SLAB)])

    @pl.when(sid == 0)
    def _():
        pltpu.sync_copy(zbuf.at[pl.ds(0, 16)],
                        acc_sh.at[pl.ds(NS * SLAB, N - NS * SLAB)])

    ov = jnp.full((16,), 1.0, jnp.float32)
    for o in range(0, KD - 15, 16):
        ones_v[pl.ds(o, 16)] = ov
    ones_v[pl.ds(KD - 16, 16)] = ov
    pltpu.sync_copy(dst_hbm.at[cid, sid], idx_v)
    plsc.subcore_barrier()

    def body(j, carry):
        pltpu.sync_copy(ones_v, acc_sh.at[idx_v.at[j]], add=True)
        return carry

    lax.fori_loop(0, CHD, body, 0)
    plsc.subcore_barrier()
    pltpu.sync_copy(acc_sh.at[pl.ds(sid * SLAB, SLAB)],
                    out_hbm.at[cid, pl.ds(sid * SLAB, SLAB)])

    @pl.when(sid == 0)
    def _():
        pltpu.sync_copy(acc_sh.at[pl.ds(NS * SLAB, N - NS * SLAB)],
                        out_hbm.at[cid, pl.ds(NS * SLAB, N - NS * SLAB)])


@functools.partial(
    pl.kernel,
    out_type=jax.ShapeDtypeStruct((N, F_HID), jnp.float32),
    mesh=_mesh,
    scratch_types=[
        pltpu.VMEM((CHR, KR), jnp.int32),
        pltpu.VMEM((CHR, KR), jnp.int32),
        pltpu.VMEM((GRP, KR, DH), jnp.float32),
        pltpu.VMEM((ZR, DH), jnp.float32),
        pltpu.VMEM_SHARED((N, DH), jnp.float32),
    ] + [pltpu.SemaphoreType.DMA] * (2 * GRP),
    compiler_params=_sc_params,
)
def _scatter_call(t2_hbm, idx2_hbm, dst_hbm, out_hbm,
                  src_v, dst_v, rows_v, zbuf, acc_sh, *sems):
    cid = lax.axis_index("c")
    sid = lax.axis_index("s")
    _zero_fill(zbuf, ZR, DH)
    for q in range(RPT // ZR):
        pltpu.sync_copy(zbuf, acc_sh.at[pl.ds(sid * RPT + q * ZR, ZR)])
    pltpu.sync_copy(idx2_hbm.at[cid, sid], src_v)
    pltpu.sync_copy(dst_hbm.at[sid], dst_v)
    plsc.subcore_barrier()

    def group(j, n):
        gathers = [
            pltpu.async_copy(t2_hbm.at[src_v.at[j + i]], rows_v.at[i], sems[i])
            for i in range(n)
        ]
        scatters = []
        for i in range(n):
            gathers[i].wait()
            scatters.append(
                pltpu.async_copy(rows_v.at[i], acc_sh.at[dst_v.at[j + i]],
                                 sems[GRP + i], add=True))
        for i in range(n):
            scatters[i].wait()

    def body(p, carry):
        group(GRP * p, GRP)
        return carry

    lax.fori_loop(0, CHR // GRP, body, 0)
    if GTAIL:
        group(CHR - GTAIL, GTAIL)
    plsc.subcore_barrier()
    pltpu.sync_copy(acc_sh.at[pl.ds(sid * RPT, RPT)],
                    out_hbm.at[pl.ds(sid * RPT, RPT), pl.ds(cid * DH, DH)])


BN = 1000


def _dinv_of(deg0_ref, deg1_ref):
    return lax.rsqrt(deg0_ref[:] + deg1_ref[:] + 1.0)


def _mm1_body(x_ref, w_ref, deg0_ref, deg1_ref, o_ref):
    dinv = _dinv_of(deg0_ref, deg1_ref)
    o_ref[:] = jnp.dot(x_ref[:], w_ref[:],
                       preferred_element_type=jnp.float32) * dinv


def _mid_body(acc_ref, t_ref, deg0_ref, deg1_ref, b_ref, w_ref, o_ref):
    dinv = _dinv_of(deg0_ref, deg1_ref)
    h = jnp.maximum(dinv * (acc_ref[:] + t_ref[:]) + b_ref[:], 0.0)
    o_ref[:] = jnp.dot(h, w_ref[:], preferred_element_type=jnp.float32) * dinv


def _final_body(acc_ref, t_ref, deg0_ref, deg1_ref, b_ref, o_ref):
    dinv = _dinv_of(deg0_ref, deg1_ref)
    o_ref[:] = jnp.maximum(dinv * (acc_ref[:] + t_ref[:]) + b_ref[:], 0.0)


_t_spec = pl.BlockSpec((BN, F_HID), lambda i: (i, 0))
_deg_spec = pl.BlockSpec((BN, 1), lambda i: (i, 0))
_bias_spec = pl.BlockSpec((1, F_HID), lambda i: (0, 0))

_mm1 = pl.pallas_call(
    _mm1_body,
    grid=(N // BN,),
    in_specs=[
        pl.BlockSpec((BN, F_IN), lambda i: (i, 0)),
        pl.BlockSpec((F_IN, F_HID), lambda i: (0, 0)),
        _deg_spec,
        _deg_spec,
    ],
    out_specs=_t_spec,
    out_shape=jax.ShapeDtypeStruct((N, F_HID), jnp.float32),
)

_mid = pl.pallas_call(
    _mid_body,
    grid=(N // BN,),
    in_specs=[
        _t_spec,
        _t_spec,
        _deg_spec,
        _deg_spec,
        _bias_spec,
        pl.BlockSpec((F_HID, F_HID), lambda i: (0, 0)),
    ],
    out_specs=_t_spec,
    out_shape=jax.ShapeDtypeStruct((N, F_HID), jnp.float32),
)

_final = pl.pallas_call(
    _final_body,
    grid=(N // BN,),
    in_specs=[_t_spec, _t_spec, _deg_spec, _deg_spec, _bias_spec],
    out_specs=_t_spec,
    out_shape=jax.ShapeDtypeStruct((N, F_HID), jnp.float32),
)


def kernel(x, edge_index, batch, W1, b1, W2, b2):
    src = edge_index[0]
    dst = edge_index[1]
    dst_deg = dst.reshape(NC, NS, CHD, KD)
    src2 = 2 * src.reshape(NS, CHR, KR)
    idx2 = jnp.stack([src2, src2 + 1])
    dst_r = dst.reshape(NS, CHR, KR)
    b1r = b1.reshape(1, F_HID)
    b2r = b2.reshape(1, F_HID)

    degp = _deg_call(dst_deg)
    deg0 = degp[0].reshape(N, 1)
    deg1 = degp[1].reshape(N, 1)
    t1 = _mm1(x, W1, deg0, deg1)
    acc1 = _scatter_call(t1.reshape(2 * N, DH), idx2, dst_r)
    t2 = _mid(acc1, t1, deg0, deg1, b1r, W2)
    acc2 = _scatter_call(t2.reshape(2 * N, DH), idx2, dst_r)
    return _final(acc2, t2, deg0, deg1, b2r)


# speedup vs baseline: 1.5043x; 1.0465x over previous
import functools

import jax
import jax.numpy as jnp
from jax import lax
from jax.experimental import pallas as pl
from jax.experimental.pallas import tpu as pltpu
from jax.experimental.pallas import tpu_sc as plsc

N = 10000
E = 160000
F_IN = 400
F_HID = 128
NC = 2
NS = 16
DH = F_HID // NC

KD = 40
CHD = E // (NC * NS * KD)
KR = 80
CHR = E // (NS * KR)
GRP = 12
GTAIL = CHR % GRP

RPT = N // NS
ZR = 125

_mesh = plsc.VectorSubcoreMesh(core_axis_name="c", subcore_axis_name="s")
_sc_params = pltpu.CompilerParams(use_tc_tiling_on_sc=False)


def _zero_fill(zbuf, nrows, width):
    zv = jnp.zeros((16,), jnp.float32)

    def zrow(r, carry):
        for c in range(width // 16):
            zbuf[r, pl.ds(c * 16, 16)] = zv
        return carry

    lax.fori_loop(0, nrows, zrow, 0)


SLAB = 624


@functools.partial(
    pl.kernel,
    out_type=jax.ShapeDtypeStruct((NC, N), jnp.float32),
    mesh=_mesh,
    scratch_types=[
        pltpu.VMEM((CHD, KD), jnp.int32),
        pltpu.VMEM((KD,), jnp.float32),
        pltpu.VMEM((SLAB,), jnp.float32),
        pltpu.VMEM_SHARED((N,), jnp.float32),
    ],
    compiler_params=_sc_params,
)
def _deg_call(dst_hbm, out_hbm, idx_v, ones_v, zbuf, acc_sh):
    cid = lax.axis_index("c")
    sid = lax.axis_index("s")
    zv = jnp.zeros((16,), jnp.float32)
    for o in range(0, SLAB, 16):
        zbuf[pl.ds(o, 16)] = zv
    pltpu.sync_copy(zbuf, acc_sh.at[pl.ds(sid * SLAB, SLAB)])

    @pl.when(sid == 0)
    def _():
        pltpu.sync_copy(zbuf.at[pl.ds(0, 16)],
                        acc_sh.at[pl.ds(NS * SLAB, N - NS * SLAB)])

    ov = jnp.full((16,), 1.0, jnp.float32)
    for o in range(0, KD - 15, 16):
        ones_v[pl.ds(o, 16)] = ov
    ones_v[pl.ds(KD - 16, 16)] = ov
    pltpu.sync_copy(dst_hbm.at[cid, sid], idx_v)
    plsc.subcore_barrier()

    def body(j, carry):
        pltpu.sync_copy(ones_v, acc_sh.at[idx_v.at[j]], add=True)
        return carry

    lax.fori_loop(0, CHD, body, 0)
    plsc.subcore_barrier()
    pltpu.sync_copy(acc_sh.at[pl.ds(sid * SLAB, SLAB)],
                    out_hbm.at[cid, pl.ds(sid * SLAB, SLAB)])

    @pl.when(sid == 0)
    def _():
        pltpu.sync_copy(acc_sh.at[pl.ds(NS * SLAB, N - NS * SLAB)],
                        out_hbm.at[cid, pl.ds(NS * SLAB, N - NS * SLAB)])


@functools.partial(
    pl.kernel,
    out_type=jax.ShapeDtypeStruct((N, F_HID), jnp.float32),
    mesh=_mesh,
    scratch_types=[
        pltpu.VMEM((CHR, KR), jnp.int32),
        pltpu.VMEM((CHR, KR), jnp.int32),
        pltpu.VMEM((GRP, KR, DH), jnp.float32),
        pltpu.VMEM((ZR, DH), jnp.float32),
        pltpu.VMEM_SHARED((N, DH), jnp.float32),
    ] + [pltpu.SemaphoreType.DMA] * (2 * GRP),
    compiler_params=_sc_params,
)
def _scatter_call(t2_hbm, idx2_hbm, dst_hbm, out_hbm,
                  src_v, dst_v, rows_v, zbuf, acc_sh, *sems):
    cid = lax.axis_index("c")
    sid = lax.axis_index("s")
    cp_src = pltpu.async_copy(idx2_hbm.at[cid, sid], src_v, sems[0])
    cp_dst = pltpu.async_copy(dst_hbm.at[sid], dst_v, sems[1])
    _zero_fill(zbuf, ZR, DH)
    for q in range(RPT // ZR):
        pltpu.sync_copy(zbuf, acc_sh.at[pl.ds(sid * RPT + q * ZR, ZR)])
    cp_src.wait()
    cp_dst.wait()
    plsc.subcore_barrier()

    def group(j, n):
        gathers = [
            pltpu.async_copy(t2_hbm.at[src_v.at[j + i]], rows_v.at[i], sems[i])
            for i in range(n)
        ]
        scatters = []
        for i in range(n):
            gathers[i].wait()
            scatters.append(
                pltpu.async_copy(rows_v.at[i], acc_sh.at[dst_v.at[j + i]],
                                 sems[GRP + i], add=True))
        for i in range(n):
            scatters[i].wait()

    def body(p, carry):
        group(GRP * p, GRP)
        return carry

    lax.fori_loop(0, CHR // GRP, body, 0)
    if GTAIL:
        group(CHR - GTAIL, GTAIL)
    plsc.subcore_barrier()
    pltpu.sync_copy(acc_sh.at[pl.ds(sid * RPT, RPT)],
                    out_hbm.at[pl.ds(sid * RPT, RPT), pl.ds(cid * DH, DH)])


BN = 1000


def _dinv_of(deg0_ref, deg1_ref):
    return lax.rsqrt(deg0_ref[:] + deg1_ref[:] + 1.0)


def _mm1_body(x_ref, w_ref, deg0_ref, deg1_ref, o_ref):
    dinv = _dinv_of(deg0_ref, deg1_ref)
    o_ref[:] = jnp.dot(x_ref[:], w_ref[:],
                       preferred_element_type=jnp.float32) * dinv


def _mid_body(acc_ref, t_ref, deg0_ref, deg1_ref, b_ref, w_ref, o_ref):
    dinv = _dinv_of(deg0_ref, deg1_ref)
    h = jnp.maximum(dinv * (acc_ref[:] + t_ref[:]) + b_ref[:], 0.0)
    o_ref[:] = jnp.dot(h, w_ref[:], preferred_element_type=jnp.float32) * dinv


def _final_body(acc_ref, t_ref, deg0_ref, deg1_ref, b_ref, o_ref):
    dinv = _dinv_of(deg0_ref, deg1_ref)
    o_ref[:] = jnp.maximum(dinv * (acc_ref[:] + t_ref[:]) + b_ref[:], 0.0)


_t_spec = pl.BlockSpec((BN, F_HID), lambda i: (i, 0))
_deg_spec = pl.BlockSpec((BN, 1), lambda i: (i, 0))
_bias_spec = pl.BlockSpec((1, F_HID), lambda i: (0, 0))

_mm1 = pl.pallas_call(
    _mm1_body,
    grid=(N // BN,),
    in_specs=[
        pl.BlockSpec((BN, F_IN), lambda i: (i, 0)),
        pl.BlockSpec((F_IN, F_HID), lambda i: (0, 0)),
        _deg_spec,
        _deg_spec,
    ],
    out_specs=_t_spec,
    out_shape=jax.ShapeDtypeStruct((N, F_HID), jnp.float32),
)

_mid = pl.pallas_call(
    _mid_body,
    grid=(N // BN,),
    in_specs=[
        _t_spec,
        _t_spec,
        _deg_spec,
        _deg_spec,
        _bias_spec,
        pl.BlockSpec((F_HID, F_HID), lambda i: (0, 0)),
    ],
    out_specs=_t_spec,
    out_shape=jax.ShapeDtypeStruct((N, F_HID), jnp.float32),
)

_final = pl.pallas_call(
    _final_body,
    grid=(N // BN,),
    in_specs=[_t_spec, _t_spec, _deg_spec, _deg_spec, _bias_spec],
    out_specs=_t_spec,
    out_shape=jax.ShapeDtypeStruct((N, F_HID), jnp.float32),
)


def kernel(x, edge_index, batch, W1, b1, W2, b2):
    src = edge_index[0]
    dst = edge_index[1]
    dst_deg = dst.reshape(NC, NS, CHD, KD)
    src2 = 2 * src.reshape(NS, CHR, KR)
    idx2 = jnp.stack([src2, src2 + 1])
    dst_r = dst.reshape(NS, CHR, KR)
    b1r = b1.reshape(1, F_HID)
    b2r = b2.reshape(1, F_HID)

    degp = _deg_call(dst_deg)
    deg0 = degp[0].reshape(N, 1)
    deg1 = degp[1].reshape(N, 1)
    t1 = _mm1(x, W1, deg0, deg1)
    acc1 = _scatter_call(t1.reshape(2 * N, DH), idx2, dst_r)
    t2 = _mid(acc1, t1, deg0, deg1, b1r, W2)
    acc2 = _scatter_call(t2.reshape(2 * N, DH), idx2, dst_r)
    return _final(acc2, t2, deg0, deg1, b2r)
